# baseline (speedup 1.0000x reference)
import jax
import jax.numpy as jnp
from jax.experimental import pallas as pl

_K = 8
_BS = 32


def _fused_body(x_ref, enc_wT_ref, enc_b_ref, dec_flat_ref, dec_b_ref,
                comp_b_ref, scale_ref, router_w_ref, expand_ref, out_ref):
    x = x_ref[...]

    latent = jnp.dot(x.astype(jnp.bfloat16), enc_wT_ref[...],
                     preferred_element_type=jnp.float32)
    latent = latent + enc_b_ref[...]

    scores = jnp.dot(x, router_w_ref[...], preferred_element_type=jnp.float32)

    neg_inf = jnp.float32(-jnp.inf)

    rem = scores
    row_max = None
    thresh = None
    for _ in range(_K):
        thresh = jnp.max(rem, axis=1, keepdims=True)
        if row_max is None:
            row_max = thresh
        rem = jnp.where(rem == thresh, neg_inf, rem)

    sel_fast = scores >= thresh
    cnt = jnp.sum(jnp.where(sel_fast, 1.0, 0.0), axis=1, keepdims=True)
    no_ties = jnp.all(cnt == jnp.float32(_K))

    def _fast(_):
        e = jnp.exp(jnp.where(sel_fast, scores - row_max, neg_inf))
        return e / jnp.sum(e, axis=1, keepdims=True)

    def _exact(_):
        colf = jax.lax.broadcasted_iota(jnp.int32, scores.shape, 1).astype(
            jnp.float32)
        big = jnp.float32(1e9)
        remaining = scores
        selected = jnp.zeros(scores.shape, dtype=jnp.bool_)
        for _k in range(_K):
            m = jnp.max(remaining, axis=1, keepdims=True)
            z = jnp.where(remaining == m, colf, big)
            pick = z == jnp.min(z, axis=1, keepdims=True)
            selected = jnp.logical_or(selected, pick)
            remaining = jnp.where(pick, neg_inf, remaining)
        e = jnp.exp(jnp.where(selected, scores - row_max, neg_inf))
        return e / jnp.sum(e, axis=1, keepdims=True)

    wfull = jax.lax.cond(no_ties, _fast, _exact, None)

    wcol = jnp.dot(wfull.astype(jnp.bfloat16), expand_ref[...],
                   preferred_element_type=jnp.float32)

    y = jnp.dot(latent.astype(jnp.bfloat16), dec_flat_ref[...],
                preferred_element_type=jnp.float32)
    y = y + dec_b_ref[...]
    out_ref[...] = y * wcol * scale_ref[0, 0] + comp_b_ref[...]


def kernel(x, enc_w, enc_b, dec, dec_b, comp_b, scale, router_w):
    T, H = x.shape
    NB, R, BS = dec.shape[0], dec.shape[1], dec.shape[2]

    enc_wT = enc_w.T.astype(jnp.bfloat16)
    dec_flat = jnp.transpose(dec, (1, 0, 2)).reshape(R, NB * BS)
    dec_flat = dec_flat.astype(jnp.bfloat16)
    dec_b_flat = dec_b.reshape(1, NB * BS)
    enc_b2 = enc_b.reshape(1, R)
    comp_b2 = comp_b.reshape(1, NB * BS)
    scale2 = jnp.reshape(scale, (1, 1)).astype(jnp.float32)
    expand = jnp.repeat(jnp.eye(NB, dtype=jnp.bfloat16), BS, axis=1)

    bT = 512 if T % 512 == 0 else T
    grid = (T // bT,)

    def tok_map(i):
        return (i, 0)

    def fixed_map(i):
        return (0, 0)

    return pl.pallas_call(
        _fused_body,
        grid=grid,
        in_specs=[
            pl.BlockSpec((bT, H), tok_map),
            pl.BlockSpec((H, R), fixed_map),
            pl.BlockSpec((1, R), fixed_map),
            pl.BlockSpec((R, NB * BS), fixed_map),
            pl.BlockSpec((1, NB * BS), fixed_map),
            pl.BlockSpec((1, NB * BS), fixed_map),
            pl.BlockSpec((1, 1), fixed_map),
            pl.BlockSpec((H, NB), fixed_map),
            pl.BlockSpec((NB, NB * BS), fixed_map),
        ],
        out_specs=pl.BlockSpec((bT, NB * BS), tok_map),
        out_shape=jax.ShapeDtypeStruct((T, NB * BS), jnp.float32),
    )(x, enc_wT, enc_b2, dec_flat, dec_b_flat, comp_b2, scale2, router_w,
      expand)


# speedup vs baseline: 1.0670x; 1.0670x over previous
import jax
import jax.numpy as jnp
from jax.experimental import pallas as pl

_K = 8
_BS = 32


def _fused_body(x_ref, enc_wT_ref, enc_b_ref, dec_flat_ref, dec_b_ref,
                comp_b_ref, scale_ref, router_w_ref, expand_ref, out_ref):
    x = x_ref[...]

    latent = jnp.dot(x.astype(jnp.bfloat16), enc_wT_ref[...],
                     preferred_element_type=jnp.float32)
    latent = latent + enc_b_ref[...]

    scores = jnp.dot(x, router_w_ref[...], preferred_element_type=jnp.float32)

    neg_inf = jnp.float32(-jnp.inf)

    rem = scores
    row_max = None
    thresh = None
    for _ in range(_K):
        thresh = jnp.max(rem, axis=1, keepdims=True)
        if row_max is None:
            row_max = thresh
        rem = jnp.where(rem == thresh, neg_inf, rem)

    sel_fast = scores >= thresh
    cnt = jnp.sum(jnp.where(sel_fast, 1.0, 0.0), axis=1, keepdims=True)
    no_ties = jnp.all(cnt == jnp.float32(_K))

    def _fast(_):
        e = jnp.exp(jnp.where(sel_fast, scores - row_max, neg_inf))
        return e / jnp.sum(e, axis=1, keepdims=True)

    def _exact(_):
        colf = jax.lax.broadcasted_iota(jnp.int32, scores.shape, 1).astype(
            jnp.float32)
        big = jnp.float32(1e9)
        remaining = scores
        selected = jnp.zeros(scores.shape, dtype=jnp.bool_)
        for _k in range(_K):
            m = jnp.max(remaining, axis=1, keepdims=True)
            z = jnp.where(remaining == m, colf, big)
            pick = z == jnp.min(z, axis=1, keepdims=True)
            selected = jnp.logical_or(selected, pick)
            remaining = jnp.where(pick, neg_inf, remaining)
        e = jnp.exp(jnp.where(selected, scores - row_max, neg_inf))
        return e / jnp.sum(e, axis=1, keepdims=True)

    wfull = jax.lax.cond(no_ties, _fast, _exact, None)

    wcol = jnp.dot(wfull.astype(jnp.bfloat16), expand_ref[...],
                   preferred_element_type=jnp.float32)

    y = jnp.dot(latent.astype(jnp.bfloat16), dec_flat_ref[...],
                preferred_element_type=jnp.float32)
    y = y + dec_b_ref[...]
    out_ref[...] = y * wcol * scale_ref[0, 0] + comp_b_ref[...]


def kernel(x, enc_w, enc_b, dec, dec_b, comp_b, scale, router_w):
    T, H = x.shape
    NB, R, BS = dec.shape[0], dec.shape[1], dec.shape[2]

    enc_wT = enc_w.T.astype(jnp.bfloat16)
    dec_flat = jnp.transpose(dec, (1, 0, 2)).reshape(R, NB * BS)
    dec_flat = dec_flat.astype(jnp.bfloat16)
    dec_b_flat = dec_b.reshape(1, NB * BS)
    enc_b2 = enc_b.reshape(1, R)
    comp_b2 = comp_b.reshape(1, NB * BS)
    scale2 = jnp.reshape(scale, (1, 1)).astype(jnp.float32)
    expand = jnp.repeat(jnp.eye(NB, dtype=jnp.bfloat16), BS, axis=1)

    bT = 1024 if T % 1024 == 0 else T
    grid = (T // bT,)

    def tok_map(i):
        return (i, 0)

    def fixed_map(i):
        return (0, 0)

    return pl.pallas_call(
        _fused_body,
        grid=grid,
        in_specs=[
            pl.BlockSpec((bT, H), tok_map),
            pl.BlockSpec((H, R), fixed_map),
            pl.BlockSpec((1, R), fixed_map),
            pl.BlockSpec((R, NB * BS), fixed_map),
            pl.BlockSpec((1, NB * BS), fixed_map),
            pl.BlockSpec((1, NB * BS), fixed_map),
            pl.BlockSpec((1, 1), fixed_map),
            pl.BlockSpec((H, NB), fixed_map),
            pl.BlockSpec((NB, NB * BS), fixed_map),
        ],
        out_specs=pl.BlockSpec((bT, NB * BS), tok_map),
        out_shape=jax.ShapeDtypeStruct((T, NB * BS), jnp.float32),
    )(x, enc_wT, enc_b2, dec_flat, dec_b_flat, comp_b2, scale2, router_w,
      expand)


# speedup vs baseline: 1.1151x; 1.0450x over previous
import jax
import jax.numpy as jnp
from jax.experimental import pallas as pl

_K = 8
_BS = 32


def _fused_body(x_ref, enc_wT_ref, enc_b_ref, dec_flat_ref, dec_b_ref,
                comp_b_ref, router_w_ref, expand_ref, out_ref):
    x = x_ref[...]

    latent = jnp.dot(x.astype(jnp.bfloat16), enc_wT_ref[...],
                     preferred_element_type=jnp.float32)
    latent = latent + enc_b_ref[...]

    scores = jnp.dot(x, router_w_ref[...], preferred_element_type=jnp.float32)

    neg_inf = jnp.float32(-jnp.inf)

    rem = scores
    row_max = None
    thresh = None
    for _ in range(_K):
        thresh = jnp.max(rem, axis=1, keepdims=True)
        if row_max is None:
            row_max = thresh
        rem = jnp.where(rem == thresh, neg_inf, rem)

    sel_fast = scores >= thresh
    cnt = jnp.sum(jnp.where(sel_fast, 1.0, 0.0), axis=1, keepdims=True)
    no_ties = jnp.all(cnt == jnp.float32(_K))

    def _fast(_):
        e = jnp.exp(jnp.where(sel_fast, scores - row_max, neg_inf))
        return e / jnp.sum(e, axis=1, keepdims=True)

    def _exact(_):
        colf = jax.lax.broadcasted_iota(jnp.int32, scores.shape, 1).astype(
            jnp.float32)
        big = jnp.float32(1e9)
        remaining = scores
        selected = jnp.zeros(scores.shape, dtype=jnp.bool_)
        for _k in range(_K):
            m = jnp.max(remaining, axis=1, keepdims=True)
            z = jnp.where(remaining == m, colf, big)
            pick = z == jnp.min(z, axis=1, keepdims=True)
            selected = jnp.logical_or(selected, pick)
            remaining = jnp.where(pick, neg_inf, remaining)
        e = jnp.exp(jnp.where(selected, scores - row_max, neg_inf))
        return e / jnp.sum(e, axis=1, keepdims=True)

    wfull = jax.lax.cond(no_ties, _fast, _exact, None)

    wcol = jnp.dot(wfull.astype(jnp.bfloat16), expand_ref[...],
                   preferred_element_type=jnp.float32)

    y = jnp.dot(latent.astype(jnp.bfloat16), dec_flat_ref[...],
                preferred_element_type=jnp.float32)
    y = y + dec_b_ref[...]
    out_ref[...] = y * wcol + comp_b_ref[...]


def kernel(x, enc_w, enc_b, dec, dec_b, comp_b, scale, router_w):
    T, H = x.shape
    NB, R, BS = dec.shape[0], dec.shape[1], dec.shape[2]

    scale_f = jnp.reshape(scale, ()).astype(jnp.float32)
    enc_wT = enc_w.T.astype(jnp.bfloat16)
    dec_flat = jnp.transpose(dec, (1, 0, 2)).reshape(R, NB * BS)
    dec_flat = (dec_flat * scale_f).astype(jnp.bfloat16)
    dec_b_flat = (dec_b.reshape(1, NB * BS) * scale_f)
    enc_b2 = enc_b.reshape(1, R)
    comp_b2 = comp_b.reshape(1, NB * BS)
    expand = jnp.repeat(jnp.eye(NB, dtype=jnp.bfloat16), BS, axis=1)

    bT = 1024 if T % 1024 == 0 else T
    grid = (T // bT,)

    def tok_map(i):
        return (i, 0)

    def fixed_map(i):
        return (0, 0)

    return pl.pallas_call(
        _fused_body,
        grid=grid,
        in_specs=[
            pl.BlockSpec((bT, H), tok_map),
            pl.BlockSpec((H, R), fixed_map),
            pl.BlockSpec((1, R), fixed_map),
            pl.BlockSpec((R, NB * BS), fixed_map),
            pl.BlockSpec((1, NB * BS), fixed_map),
            pl.BlockSpec((1, NB * BS), fixed_map),
            pl.BlockSpec((H, NB), fixed_map),
            pl.BlockSpec((NB, NB * BS), fixed_map),
        ],
        out_specs=pl.BlockSpec((bT, NB * BS), tok_map),
        out_shape=jax.ShapeDtypeStruct((T, NB * BS), jnp.float32),
    )(x, enc_wT, enc_b2, dec_flat, dec_b_flat, comp_b2, router_w, expand)
